# baseline (speedup 1.0000x reference)
import jax
import jax.numpy as jnp
from jax import lax
from jax.experimental import pallas as pl
from jax.experimental.pallas import tpu as pltpu
from jax.experimental.pallas import tpu_sc as plsc

VOCAB = 1_000_000
EMB = 64
N_SEQ = 4096
SEQ_LEN = 200
LANES = 128

_info = plsc.get_sparse_core_info()
NC, NS = _info.num_cores, _info.num_subcores
NW = NC * NS


def _body(idxT_hbm, t2_hbm, out_hbm, idxcol, k0, k1, p0, p1, g0, g1, b0, b1,
          gsem, wsem):
    wid = lax.axis_index("s") * NC + lax.axis_index("c")
    s0 = wid * LANES
    pltpu.sync_copy(idxT_hbm.at[:, pl.ds(s0, LANES)], idxcol)

    ktmp = (k0, k1)
    pcol = (p0, p1)
    G = (g0, g1)
    B = (b0, b1)
    iota = lax.iota(jnp.int32, 16)
    rows = [iota + (16 * g) for g in range(8)]

    def build(t, slot):
        for g in range(8):
            v = idxcol[t, pl.ds(16 * g, 16)]
            ktmp[slot][pl.ds(16 * g, 16)] = lax.shift_right_logical(v, 1)
            pcol[slot][pl.ds(16 * g, 16)] = lax.shift_left(
                lax.bitwise_and(v, 1), 6)

    def issue_gather(slot):
        pltpu.async_copy(t2_hbm.at[ktmp[slot]], G[slot], gsem.at[slot])

    def drain_gather(slot):
        pltpu.make_async_copy(
            t2_hbm.at[pl.ds(0, LANES)], G[slot], gsem.at[slot]).wait()

    def select(slot):
        pbases = [pcol[slot][pl.ds(16 * g, 16)] for g in range(8)]

        for e in range(EMB):
            for g in range(8):
                col = pbases[g] + e
                x = plsc.load_gather(G[slot], [rows[g], col])
                B[slot][e, pl.ds(16 * g, 16)] = x

    def issue_write(t, slot):
        pltpu.async_copy(
            B[slot], out_hbm.at[t, :, pl.ds(s0, LANES)], wsem.at[slot])

    def wait_write(t, slot):
        pltpu.make_async_copy(
            B[slot], out_hbm.at[t, :, pl.ds(s0, LANES)], wsem.at[slot]).wait()

    build(0, 0)
    issue_gather(0)

    @pl.loop(0, SEQ_LEN // 2)
    def t_loop(ti):
        for bslot in range(2):
            t = 2 * ti + bslot
            nslot = 1 - bslot

            @pl.when(t + 1 < SEQ_LEN)
            def _prefetch():
                build(t + 1, nslot)
                issue_gather(nslot)

            drain_gather(bslot)

            @pl.when(t >= 2)
            def _reclaim():
                wait_write(t - 2, bslot)

            select(bslot)
            issue_write(t, bslot)

    wait_write(SEQ_LEN - 2, 0)
    wait_write(SEQ_LEN - 1, 1)


@jax.jit
def _embed(idxT, t2):
    mesh = plsc.VectorSubcoreMesh(core_axis_name="c", subcore_axis_name="s")
    k = pl.kernel(
        _body,
        out_type=jax.ShapeDtypeStruct((SEQ_LEN, EMB, N_SEQ), jnp.float32),
        mesh=mesh,
        scratch_types=[
            pltpu.VMEM((SEQ_LEN, LANES), jnp.int32),
            pltpu.VMEM((LANES,), jnp.int32),
            pltpu.VMEM((LANES,), jnp.int32),
            pltpu.VMEM((LANES,), jnp.int32),
            pltpu.VMEM((LANES,), jnp.int32),
            pltpu.VMEM((LANES, LANES), jnp.float32),
            pltpu.VMEM((LANES, LANES), jnp.float32),
            pltpu.VMEM((EMB, LANES), jnp.float32),
            pltpu.VMEM((EMB, LANES), jnp.float32),
            pltpu.SemaphoreType.DMA((2,)),
            pltpu.SemaphoreType.DMA((2,)),
        ],
        compiler_params=pltpu.CompilerParams(
            use_tc_tiling_on_sc=True, needs_layout_passes=False),
    )
    return k(idxT, t2)


def kernel(input_vars, table):
    idxT = input_vars.astype(jnp.int32).T
    t2 = table.reshape(500000, 128)
    out2 = _embed(idxT, t2)
    return out2.transpose(2, 0, 1)


# speedup vs baseline: 1.5784x; 1.5784x over previous
import jax
import jax.numpy as jnp
from jax import lax
from jax.experimental import pallas as pl
from jax.experimental.pallas import tpu as pltpu
from jax.experimental.pallas import tpu_sc as plsc

VOCAB = 1_000_000
EMB = 64
N_SEQ = 4096
SEQ_LEN = 200

_info = plsc.get_sparse_core_info()
NC, NS = _info.num_cores, _info.num_subcores
NW = NC * NS
SEQ_PER_W = N_SEQ // NW
SPLITS = ((0, 128), (128, 72))
NBUF = 4
AHEAD = NBUF - 1


def _body(idx_hbm, table_hbm, out_hbm, idx_v, buf, gsem, wsem):
    wid = lax.axis_index("s") * NC + lax.axis_index("c")
    seq0 = wid * SEQ_PER_W
    pltpu.sync_copy(idx_hbm.at[pl.ds(seq0, SEQ_PER_W)], idx_v)

    def issue_gathers(s, b):
        for off, size in SPLITS:
            pltpu.async_copy(
                table_hbm.at[idx_v.at[s, pl.ds(off, size)]],
                buf.at[b, pl.ds(off, size)],
                gsem.at[b],
            )

    def drain_gathers(b):
        pltpu.make_async_copy(
            table_hbm.at[pl.ds(0, SEQ_LEN)], buf.at[b], gsem.at[b]
        ).wait()

    def issue_write(s, b):
        pltpu.async_copy(buf.at[b], out_hbm.at[seq0 + s], wsem.at[b])

    def wait_write(s, b):
        pltpu.make_async_copy(
            buf.at[b], out_hbm.at[seq0 + s], wsem.at[b]
        ).wait()

    for s in range(AHEAD):
        issue_gathers(s, s)

    @pl.loop(0, SEQ_PER_W)
    def seq_loop(s):
        b = lax.rem(s, NBUF)
        nxt = s + AHEAD

        @pl.when(nxt < SEQ_PER_W)
        def _issue_next():
            nb = lax.rem(nxt, NBUF)

            @pl.when(s >= 1)
            def _wait_prev_write():
                wait_write(s - 1, nb)

            issue_gathers(nxt, nb)

        drain_gathers(b)
        issue_write(s, b)

    @pl.loop(SEQ_PER_W - AHEAD, SEQ_PER_W)
    def tail_loop(s):
        wait_write(s, lax.rem(s, NBUF))


@jax.jit
def _embed(idx, table):
    mesh = plsc.VectorSubcoreMesh(core_axis_name="c", subcore_axis_name="s")
    k = pl.kernel(
        _body,
        out_type=jax.ShapeDtypeStruct((N_SEQ, SEQ_LEN, EMB), jnp.float32),
        mesh=mesh,
        scratch_types=[
            pltpu.VMEM((SEQ_PER_W, SEQ_LEN), jnp.int32),
            pltpu.VMEM((NBUF, SEQ_LEN, EMB), jnp.float32),
            pltpu.SemaphoreType.DMA((NBUF,)),
            pltpu.SemaphoreType.DMA((NBUF,)),
        ],
        compiler_params=pltpu.CompilerParams(use_tc_tiling_on_sc=False),
    )
    return k(idx, table)


def kernel(input_vars, table):
    return _embed(input_vars.astype(jnp.int32), table)
